# baseline (speedup 1.0000x reference)
import jax
import jax.numpy as jnp
from jax.experimental import pallas as pl
from jax.experimental.pallas import tpu as pltpu

_H = 256
_SPLIT = 4
_BLK = _H * _SPLIT


def _gate_kernel(*refs):
    x_refs = refs[:_SPLIT]
    w_ref, b_ref, o_ref = refs[_SPLIT:]
    for k, x_ref in enumerate(x_refs):
        s = jax.lax.dot_general(
            w_ref[...],
            x_ref[...],
            dimension_numbers=(((1,), (1,)), ((), ())),
            preferred_element_type=jnp.float32,
        )
        s = s + b_ref[0]
        m = jnp.max(s, axis=0, keepdims=True)
        e = jnp.exp(s - m)
        o = e / jnp.sum(e, axis=0, keepdims=True)
        o_ref[k * _H:(k + 1) * _H, :] = o.T


def kernel(x, W, train, active_experts):
    n_tok, d = x.shape
    n_exp = W.shape[0]
    batch = active_experts.shape[0]
    seq = n_tok // batch
    blocks_per_batch = seq // _BLK
    bias = jnp.where(active_experts > 0, 0.0, -jnp.inf).astype(jnp.float32)
    bias = bias.reshape(batch, n_exp, 1)

    def x_spec(k):
        return pl.BlockSpec((_H, d), lambda i, k=k: (_SPLIT * i + k, 0))

    out = pl.pallas_call(
        _gate_kernel,
        grid=(n_tok // _BLK,),
        in_specs=[x_spec(k) for k in range(_SPLIT)] + [
            pl.BlockSpec((n_exp, d), lambda i: (0, 0)),
            pl.BlockSpec((1, n_exp, 1), lambda i: (i // blocks_per_batch, 0, 0)),
        ],
        out_specs=pl.BlockSpec((_BLK, n_exp), lambda i: (i, 0)),
        out_shape=jax.ShapeDtypeStruct((n_tok, n_exp), jnp.float32),
        compiler_params=pltpu.CompilerParams(dimension_semantics=("parallel",)),
    )(*([x] * _SPLIT), W, bias)
    return out.reshape(batch, seq, n_exp)


# speedup vs baseline: 1.0027x; 1.0027x over previous
import jax
import jax.numpy as jnp
from jax.experimental import pallas as pl
from jax.experimental.pallas import tpu as pltpu

_H = 256
_SPLIT = 8
_BLK = _H * _SPLIT


def _gate_kernel(*refs):
    x_refs = refs[:_SPLIT]
    w_ref, b_ref, o_ref = refs[_SPLIT:]
    for k, x_ref in enumerate(x_refs):
        s = jax.lax.dot_general(
            w_ref[...],
            x_ref[...],
            dimension_numbers=(((1,), (1,)), ((), ())),
            preferred_element_type=jnp.float32,
        )
        s = s + b_ref[0]
        m = jnp.max(s, axis=0, keepdims=True)
        e = jnp.exp(s - m)
        o = e / jnp.sum(e, axis=0, keepdims=True)
        o_ref[k * _H:(k + 1) * _H, :] = o.T


def kernel(x, W, train, active_experts):
    n_tok, d = x.shape
    n_exp = W.shape[0]
    batch = active_experts.shape[0]
    seq = n_tok // batch
    blocks_per_batch = seq // _BLK
    bias = jnp.where(active_experts > 0, 0.0, -jnp.inf).astype(jnp.float32)
    bias = bias.reshape(batch, n_exp, 1)

    def x_spec(k):
        return pl.BlockSpec((_H, d), lambda i, k=k: (_SPLIT * i + k, 0))

    out = pl.pallas_call(
        _gate_kernel,
        grid=(n_tok // _BLK,),
        in_specs=[x_spec(k) for k in range(_SPLIT)] + [
            pl.BlockSpec((n_exp, d), lambda i: (0, 0)),
            pl.BlockSpec((1, n_exp, 1), lambda i: (i // blocks_per_batch, 0, 0)),
        ],
        out_specs=pl.BlockSpec((_BLK, n_exp), lambda i: (i, 0)),
        out_shape=jax.ShapeDtypeStruct((n_tok, n_exp), jnp.float32),
        compiler_params=pltpu.CompilerParams(dimension_semantics=("parallel",)),
    )(*([x] * _SPLIT), W, bias)
    return out.reshape(batch, seq, n_exp)


# speedup vs baseline: 1.0077x; 1.0050x over previous
import jax
import jax.numpy as jnp
from jax.experimental import pallas as pl
from jax.experimental.pallas import tpu as pltpu

_H = 256
_SPLIT = 8
_BLK = _H * _SPLIT


def _gate_kernel(*refs):
    x_refs = refs[:_SPLIT]
    w_ref, b_ref, o_ref = refs[_SPLIT:]
    for k, x_ref in enumerate(x_refs):
        s = jax.lax.dot_general(
            w_ref[...].astype(jnp.bfloat16),
            x_ref[...].astype(jnp.bfloat16),
            dimension_numbers=(((1,), (1,)), ((), ())),
            preferred_element_type=jnp.float32,
        )
        s = s + b_ref[0]
        m = jnp.max(s, axis=0, keepdims=True)
        e = jnp.exp(s - m)
        o = e / jnp.sum(e, axis=0, keepdims=True)
        o_ref[k * _H:(k + 1) * _H, :] = o.T


def kernel(x, W, train, active_experts):
    n_tok, d = x.shape
    n_exp = W.shape[0]
    batch = active_experts.shape[0]
    seq = n_tok // batch
    blocks_per_batch = seq // _BLK
    bias = jnp.where(active_experts > 0, 0.0, -jnp.inf).astype(jnp.float32)
    bias = bias.reshape(batch, n_exp, 1)

    def x_spec(k):
        return pl.BlockSpec((_H, d), lambda i, k=k: (_SPLIT * i + k, 0))

    out = pl.pallas_call(
        _gate_kernel,
        grid=(n_tok // _BLK,),
        in_specs=[x_spec(k) for k in range(_SPLIT)] + [
            pl.BlockSpec((n_exp, d), lambda i: (0, 0)),
            pl.BlockSpec((1, n_exp, 1), lambda i: (i // blocks_per_batch, 0, 0)),
        ],
        out_specs=pl.BlockSpec((_BLK, n_exp), lambda i: (i, 0)),
        out_shape=jax.ShapeDtypeStruct((n_tok, n_exp), jnp.float32),
        compiler_params=pltpu.CompilerParams(dimension_semantics=("parallel",)),
    )(*([x] * _SPLIT), W, bias)
    return out.reshape(batch, seq, n_exp)
